# baseline (speedup 1.0000x reference)
import jax
import jax.numpy as jnp
from jax.experimental import pallas as pl
from jax.experimental.pallas import tpu as pltpu

S = 1024
N = 512
LAT = 27

BLK = 256
RBLK = 256
NMSG = S // BLK
NROW = (2 * S) // RBLK
STEPS = NMSG + 3 * NROW + 1

_INV = 1.0 / LAT


def _floordiv(xf, _):
    return jnp.floor((xf + 0.5) * _INV)


def _body(cell_ref, cs_ref, ns_ref, nbr_ref,
          w1_ref, b1_ref, wl_ref, bl_ref, wf2_ref, bf2_ref,
          wd_ref, bd_ref, wg_ref, bg_ref,
          out_ref,
          fm_s, aggf_s, inl_s, ind_s, inf_s,
          accl_s, accf_s, accd_s, sc_s):
    i = pl.program_id(0)

    @pl.when(i == 0)
    def _setup():
        idx = nbr_ref[...].astype(jnp.float32)
        q1 = _floordiv(idx, LAT)
        z = idx - LAT * q1
        q2 = _floordiv(q1, LAT)
        y = q1 - LAT * q2
        x = q2
        ci = cell_ref[0, 0].astype(jnp.float32)
        cq1 = jnp.floor((ci + 0.5) * _INV)
        cz = ci - LAT * cq1
        cq2 = jnp.floor((cq1 + 0.5) * _INV)
        cy = cq1 - LAT * cq2
        cx = cq2
        d2 = (x - cx) ** 2 + (y - cy) ** 2 + (z - cz) ** 2
        lm = (d2 < 3.5).astype(jnp.float32)
        dm = (d2 > 24.5).astype(jnp.float32)
        fm = (1.0 - lm) * (1.0 - dm)
        fm_s[...] = fm

        cnt_l = jnp.sum(lm)
        cnt_d = jnp.sum(dm)
        cnt_f = jnp.sum(fm)

        masks = jnp.concatenate(
            [lm, dm, jnp.full((1, N), 1.0 / N, jnp.float32)], axis=0)
        aggs = jnp.dot(masks, ns_ref[...],
                       preferred_element_type=jnp.float32)
        agg_l = aggs[0:1] * (1.0 / jnp.maximum(cnt_l, 1.0))
        agg_d = aggs[1:2] * (1.0 / jnp.maximum(cnt_d, 1.0))
        mean_ns = aggs[2:3]

        cs = cs_ref[...]
        inl_s[...] = jnp.concatenate([cs, agg_l], axis=1)
        ind_s[...] = jnp.concatenate([cs, agg_d], axis=1)

        glog = jnp.dot(jnp.concatenate([cs, mean_ns], axis=1), wg_ref[...],
                       preferred_element_type=jnp.float32) + bg_ref[...]
        glog = glog - jnp.max(glog)
        ge = jnp.exp(glog)
        g = ge / jnp.sum(ge)
        sc_s[0] = g[0, 0]
        sc_s[1] = g[0, 1]
        sc_s[2] = g[0, 2]
        sc_s[3] = (cnt_l > 0.0).astype(jnp.float32)
        sc_s[4] = (cnt_f > 0.0).astype(jnp.float32)
        sc_s[5] = (cnt_d > 0.0).astype(jnp.float32)
        sc_s[6] = 1.0 / jnp.maximum(cnt_f, 1.0)

        zero = jnp.zeros((1, S), jnp.float32)
        aggf_s[...] = zero
        accl_s[...] = zero
        accf_s[...] = zero
        accd_s[...] = zero

    @pl.when(i < NMSG)
    def _msg():
        msg = jnp.tanh(jnp.dot(ns_ref[...], w1_ref[...],
                               preferred_element_type=jnp.float32) + b1_ref[...])
        aggf_s[:, pl.ds(i * BLK, BLK)] = jnp.dot(
            fm_s[...], msg, preferred_element_type=jnp.float32)

    @pl.when(jnp.logical_and(i >= NMSG, i < NMSG + NROW))
    def _local():
        r = i - NMSG
        part = inl_s[:, pl.ds(r * RBLK, RBLK)]
        accl_s[...] += jnp.dot(part, wl_ref[...],
                               preferred_element_type=jnp.float32)

    @pl.when(jnp.logical_and(i >= NMSG + NROW, i < NMSG + 2 * NROW))
    def _dist():
        r = i - (NMSG + NROW)
        part = ind_s[:, pl.ds(r * RBLK, RBLK)]
        accd_s[...] += jnp.dot(part, wd_ref[...],
                               preferred_element_type=jnp.float32)

    @pl.when(i == NMSG + 2 * NROW)
    def _mkfunc():
        inf_s[...] = jnp.concatenate(
            [cs_ref[...], aggf_s[...] * sc_s[6]], axis=1)

    @pl.when(jnp.logical_and(i >= NMSG + 2 * NROW, i < NMSG + 3 * NROW))
    def _func():
        r = i - (NMSG + 2 * NROW)
        part = inf_s[:, pl.ds(r * RBLK, RBLK)]
        accf_s[...] += jnp.dot(part, wf2_ref[...],
                               preferred_element_type=jnp.float32)

    @pl.when(i == STEPS - 1)
    def _final():
        lo = jnp.tanh(accl_s[...] + bl_ref[...]) * (sc_s[0] * sc_s[3])
        fo = jnp.tanh(accf_s[...] + bf2_ref[...]) * (sc_s[1] * sc_s[4])
        do = jnp.tanh(accd_s[...] + bd_ref[...]) * (sc_s[2] * sc_s[5])
        out_ref[...] = lo + fo + do


def _mkcall():
    last = STEPS - 1

    def w1_map(i):
        return (0, jnp.minimum(i, NMSG - 1))

    def wl_map(i):
        return (jnp.clip(i - NMSG, 0, NROW - 1), 0)

    def wd_map(i):
        return (jnp.clip(i - (NMSG + NROW), 0, NROW - 1), 0)

    def wf2_map(i):
        return (jnp.clip(i - (NMSG + 2 * NROW), 0, NROW - 1), 0)

    full = lambda i: (0, 0)

    return pl.pallas_call(
        _body,
        grid=(STEPS,),
        in_specs=[
            pl.BlockSpec((1, 1), full, memory_space=pltpu.SMEM),
            pl.BlockSpec((1, S), full),
            pl.BlockSpec((N, S), full),
            pl.BlockSpec((1, N), full),
            pl.BlockSpec((S, BLK), w1_map),
            pl.BlockSpec((1, BLK), lambda i: (0, jnp.minimum(i, NMSG - 1))),
            pl.BlockSpec((RBLK, S), wl_map),
            pl.BlockSpec((1, S), full),
            pl.BlockSpec((RBLK, S), wf2_map),
            pl.BlockSpec((1, S), full),
            pl.BlockSpec((RBLK, S), wd_map),
            pl.BlockSpec((1, S), full),
            pl.BlockSpec((2 * S, 3), full),
            pl.BlockSpec((1, 3), full),
        ],
        out_specs=pl.BlockSpec((1, S), full),
        out_shape=jax.ShapeDtypeStruct((1, S), jnp.float32),
        scratch_shapes=[
            pltpu.VMEM((1, N), jnp.float32),
            pltpu.VMEM((1, S), jnp.float32),
            pltpu.VMEM((1, 2 * S), jnp.float32),
            pltpu.VMEM((1, 2 * S), jnp.float32),
            pltpu.VMEM((1, 2 * S), jnp.float32),
            pltpu.VMEM((1, S), jnp.float32),
            pltpu.VMEM((1, S), jnp.float32),
            pltpu.VMEM((1, S), jnp.float32),
            pltpu.SMEM((8,), jnp.float32),
        ],
        compiler_params=pltpu.CompilerParams(
            dimension_semantics=("arbitrary",),
        ),
    )


def kernel(current_state, neighbor_states, cell_idx, neighbor_indices,
           W_local, b_local, W_func1, b_func1, W_func2, b_func2,
           W_dist, b_dist, W_gate, b_gate):
    cell = jnp.asarray(cell_idx, jnp.int32).reshape(1, 1)
    cs = current_state.reshape(1, S)
    nbr = jnp.asarray(neighbor_indices, jnp.int32).reshape(1, N)
    out = _mkcall()(
        cell, cs, neighbor_states, nbr,
        W_func1, b_func1.reshape(1, S),
        W_local, b_local.reshape(1, S),
        W_func2, b_func2.reshape(1, S),
        W_dist, b_dist.reshape(1, S),
        W_gate, b_gate.reshape(1, 3),
    )
    return out.reshape(S)


# speedup vs baseline: 1.8970x; 1.8970x over previous
import jax
import jax.numpy as jnp
from jax.experimental import pallas as pl
from jax.experimental.pallas import tpu as pltpu

S = 1024
N = 512
LAT = 27
NCHUNK = 4
CROWS = (2 * S) // NCHUNK

_INV = 1.0 / LAT


def _body(cell_ref, cs_ref, nbr_ref, bg_ref, wg_ref,
          b1_ref, bl_ref, bf2_ref, bd_ref,
          ns_hbm, w1_hbm, wl_hbm, wd_hbm, wf2_hbm,
          out_ref,
          ns_v, w1_v, wl_v, wd_v, wf2_v, sems):
    ns_cp = pltpu.make_async_copy(ns_hbm, ns_v, sems.at[0])
    ns_cp.start()
    w1_cp = pltpu.make_async_copy(w1_hbm, w1_v, sems.at[1])
    w1_cp.start()
    def start_chunks(src, dst, sem_base):
        cps = []
        for c in range(NCHUNK):
            cp = pltpu.make_async_copy(src.at[pl.ds(c * CROWS, CROWS), :],
                                       dst.at[pl.ds(c * CROWS, CROWS), :],
                                       sems.at[sem_base + c])
            cp.start()
            cps.append(cp)
        return cps

    wl_cps = start_chunks(wl_hbm, wl_v, 2)
    wd_cps = start_chunks(wd_hbm, wd_v, 2 + NCHUNK)
    wf2_cps = start_chunks(wf2_hbm, wf2_v, 2 + 2 * NCHUNK)

    idx = nbr_ref[...].astype(jnp.float32)
    q1 = jnp.floor((idx + 0.5) * _INV)
    z = idx - LAT * q1
    q2 = jnp.floor((q1 + 0.5) * _INV)
    y = q1 - LAT * q2
    x = q2
    ci = cell_ref[0, 0].astype(jnp.float32)
    cq1 = jnp.floor((ci + 0.5) * _INV)
    cz = ci - LAT * cq1
    cq2 = jnp.floor((cq1 + 0.5) * _INV)
    cy = cq1 - LAT * cq2
    cx = cq2
    d2 = (x - cx) ** 2 + (y - cy) ** 2 + (z - cz) ** 2
    lm = (d2 < 3.5).astype(jnp.float32)
    dm = (d2 > 24.5).astype(jnp.float32)
    fm = (1.0 - lm) * (1.0 - dm)
    cnt_l = jnp.sum(lm)
    cnt_d = jnp.sum(dm)
    cnt_f = jnp.sum(fm)
    flag_l = (cnt_l > 0.0).astype(jnp.float32)
    flag_d = (cnt_d > 0.0).astype(jnp.float32)
    flag_f = (cnt_f > 0.0).astype(jnp.float32)
    cs = cs_ref[...]

    ns_cp.wait()
    masks = jnp.concatenate(
        [lm, dm, jnp.full((1, N), 1.0 / N, jnp.float32)], axis=0)
    aggs = jnp.dot(masks, ns_v[...], preferred_element_type=jnp.float32)
    agg_l = aggs[0:1] * (1.0 / jnp.maximum(cnt_l, 1.0))
    agg_d = aggs[1:2] * (1.0 / jnp.maximum(cnt_d, 1.0))
    mean_ns = aggs[2:3]
    in_l = jnp.concatenate([cs, agg_l], axis=1)
    in_d = jnp.concatenate([cs, agg_d], axis=1)

    glog = jnp.dot(jnp.concatenate([cs, mean_ns], axis=1), wg_ref[...],
                   preferred_element_type=jnp.float32) + bg_ref[...]
    glog = glog - jnp.max(glog)
    ge = jnp.exp(glog)
    g = ge / jnp.sum(ge)

    w1_cp.wait()
    msg = jnp.tanh(jnp.dot(ns_v[...], w1_v[...],
                           preferred_element_type=jnp.float32) + b1_ref[...])
    agg_f = jnp.dot(fm, msg, preferred_element_type=jnp.float32)
    agg_f = agg_f * (1.0 / jnp.maximum(cnt_f, 1.0))
    in_f = jnp.concatenate([cs, agg_f], axis=1)

    def chunked_matvec(inp, w_v, cps):
        acc = jnp.zeros((1, S), jnp.float32)
        for c in range(NCHUNK):
            cps[c].wait()
            acc = acc + jnp.dot(inp[:, c * CROWS:(c + 1) * CROWS],
                                w_v[c * CROWS:(c + 1) * CROWS, :],
                                preferred_element_type=jnp.float32)
        return acc

    pre_l = chunked_matvec(in_l, wl_v, wl_cps)
    pre_d = chunked_matvec(in_d, wd_v, wd_cps)
    pre_f = chunked_matvec(in_f, wf2_v, wf2_cps)

    lo = jnp.tanh(pre_l + bl_ref[...]) * (g[0, 0] * flag_l)
    fo = jnp.tanh(pre_f + bf2_ref[...]) * (g[0, 1] * flag_f)
    do = jnp.tanh(pre_d + bd_ref[...]) * (g[0, 2] * flag_d)
    out_ref[...] = lo + fo + do


def _mkcall():
    vmem = lambda: pl.BlockSpec(memory_space=pltpu.MemorySpace.VMEM)
    hbm = lambda: pl.BlockSpec(memory_space=pltpu.MemorySpace.HBM)
    return pl.pallas_call(
        _body,
        in_specs=[
            pl.BlockSpec(memory_space=pltpu.MemorySpace.SMEM),
            vmem(),
            vmem(),
            vmem(),
            vmem(),
            vmem(),
            vmem(),
            vmem(),
            vmem(),
            hbm(),
            hbm(),
            hbm(),
            hbm(),
            hbm(),
        ],
        out_specs=pl.BlockSpec(memory_space=pltpu.MemorySpace.VMEM),
        out_shape=jax.ShapeDtypeStruct((1, S), jnp.float32),
        scratch_shapes=[
            pltpu.VMEM((N, S), jnp.float32),
            pltpu.VMEM((S, S), jnp.float32),
            pltpu.VMEM((2 * S, S), jnp.float32),
            pltpu.VMEM((2 * S, S), jnp.float32),
            pltpu.VMEM((2 * S, S), jnp.float32),
            pltpu.SemaphoreType.DMA((2 + 3 * NCHUNK,)),
        ],
    )


def kernel(current_state, neighbor_states, cell_idx, neighbor_indices,
           W_local, b_local, W_func1, b_func1, W_func2, b_func2,
           W_dist, b_dist, W_gate, b_gate):
    cell = jnp.asarray(cell_idx, jnp.int32).reshape(1, 1)
    cs = current_state.reshape(1, S)
    nbr = jnp.asarray(neighbor_indices, jnp.int32).reshape(1, N)
    out = _mkcall()(
        cell, cs, nbr, b_gate.reshape(1, 3), W_gate,
        b_func1.reshape(1, S), b_local.reshape(1, S),
        b_func2.reshape(1, S), b_dist.reshape(1, S),
        neighbor_states, W_func1, W_local, W_dist, W_func2,
    )
    return out.reshape(S)
